# baseline (speedup 1.0000x reference)
import functools

import jax
import jax.numpy as jnp
from jax import lax
from jax.experimental import pallas as pl
from jax.experimental.pallas import tpu as pltpu
from jax.experimental.pallas import tpu_sc as plsc

_INFO = plsc.get_sparse_core_info()
_NC, _NS, _L = _INFO.num_cores, _INFO.num_subcores, _INFO.num_lanes
_NW = _NC * _NS

_B = 16384
_D = 64
_F = 3 * _D
_TR = 46
_OROWS = _B // _NW
_BRW = 128
_NBLK = _OROWS // _BRW


def _sc_body(idx_hbm, table_hbm, out_hbm, idx_v,
             out_v0, out_v1, out_v2, out_v3,
             table_v, isem, wsem):
    out_bufs = (out_v0, out_v1, out_v2, out_v3)

    wid = lax.axis_index("s") * _NC + lax.axis_index("c")
    b0 = wid * _OROWS

    pltpu.sync_copy(table_hbm, table_v)

    lane = lax.iota(jnp.int32, _L)
    zero = lane * 0
    csplat = (zero, zero + 1, zero + 2)
    offs = (-1, 5, 42)

    def fetch(blk):
        return pltpu.async_copy(
            idx_hbm.at[pl.ds(b0 + blk * _BRW, _BRW)], idx_v, isem
        )

    def make_body(idx_v, out_v):
        def body(g, carry):
            rvec = lane + g * _L
            tb = [
                plsc.load_gather(idx_v, [rvec, csplat[c]]) + offs[c]
                for c in range(3)
            ]
            for c in range(3):
                for d in range(_D):
                    val = plsc.load_gather(table_v, [tb[c], zero + d])
                    out_v[c * _D + d, pl.ds(g * _L, _L)] = val
            return carry
        return body

    bodies = tuple(make_body(idx_v, out_bufs[blk]) for blk in range(_NBLK))

    writes = []
    pending = fetch(0)
    for blk in range(_NBLK):
        pending.wait()
        lax.fori_loop(0, _BRW // _L, bodies[blk], 0)
        if blk + 1 < _NBLK:
            pending = fetch(blk + 1)
        writes.append(
            pltpu.async_copy(
                out_bufs[blk],
                out_hbm.at[:, pl.ds(b0 + blk * _BRW, _BRW)],
                wsem,
            )
        )
    for w in writes:
        w.wait()


@functools.partial(jax.jit)
def kernel(inputs, embed_0, embed_1, embed_2):
    table = jnp.concatenate([embed_0, embed_1, embed_2], axis=0)

    mesh = plsc.VectorSubcoreMesh(core_axis_name="c", subcore_axis_name="s")
    out = pl.kernel(
        _sc_body,
        mesh=mesh,
        compiler_params=pltpu.CompilerParams(
            use_tc_tiling_on_sc=True, needs_layout_passes=False
        ),
        out_type=jax.ShapeDtypeStruct((_F, _B), jnp.float32),
        scratch_types=[
            pltpu.VMEM((_BRW, 3), jnp.int32),
            pltpu.VMEM((_F, _BRW), jnp.float32),
            pltpu.VMEM((_F, _BRW), jnp.float32),
            pltpu.VMEM((_F, _BRW), jnp.float32),
            pltpu.VMEM((_F, _BRW), jnp.float32),
            pltpu.VMEM((_TR, _D), jnp.float32),
            pltpu.SemaphoreType.DMA,
            pltpu.SemaphoreType.DMA,
        ],
    )(inputs, table)
    return out.T


# speedup vs baseline: 1.0938x; 1.0938x over previous
import functools

import jax
import jax.numpy as jnp
from jax import lax
from jax.experimental import pallas as pl
from jax.experimental.pallas import tpu as pltpu
from jax.experimental.pallas import tpu_sc as plsc

_INFO = plsc.get_sparse_core_info()
_NC, _NS, _L = _INFO.num_cores, _INFO.num_subcores, _INFO.num_lanes
_NW = _NC * _NS

_B = 16384
_D = 64
_F = 3 * _D
_TR = 46
_OROWS = _B // _NW
_BRW = 128
_NBLK = _OROWS // _BRW


def _sc_body(idx_hbm, table_hbm, out_hbm, idx_v0, idx_v1, out_v0, out_v1,
             table_v, isem0, isem1, wsem0, wsem1):
    idx_bufs = (idx_v0, idx_v1)
    out_bufs = (out_v0, out_v1)
    isems = (isem0, isem1)
    wsems = (wsem0, wsem1)

    wid = lax.axis_index("s") * _NC + lax.axis_index("c")
    b0 = wid * _OROWS

    pltpu.sync_copy(table_hbm, table_v)

    lane = lax.iota(jnp.int32, _L)
    zero = lane * 0
    csplat = (zero, zero + 1, zero + 2)
    offs = (-1, 5, 42)

    def fetch(blk):
        p = blk % 2
        return pltpu.async_copy(
            idx_hbm.at[pl.ds(b0 + blk * _BRW, _BRW)], idx_bufs[p], isems[p]
        )

    def make_body(idx_v, out_v):
        def body(g, carry):
            rvec = lane + g * _L
            tb = [
                plsc.load_gather(idx_v, [rvec, csplat[c]]) + offs[c]
                for c in range(3)
            ]
            for c in range(3):
                for d in range(_D):
                    val = plsc.load_gather(table_v, [tb[c], zero + d])
                    out_v[c * _D + d, pl.ds(g * _L, _L)] = val
            return carry
        return body

    bodies = (make_body(idx_v0, out_v0), make_body(idx_v1, out_v1))

    fetches = [fetch(0), fetch(1)]
    writes = [None, None]
    for blk in range(_NBLK):
        p = blk % 2
        fetches[blk].wait()
        if blk + 2 < _NBLK:
            fetches.append(fetch(blk + 2))
        if writes[p] is not None:
            writes[p].wait()
        lax.fori_loop(0, _BRW // _L, bodies[p], 0)
        writes[p] = pltpu.async_copy(
            out_bufs[p],
            out_hbm.at[:, pl.ds(b0 + blk * _BRW, _BRW)],
            wsems[p],
        )
    writes[0].wait()
    writes[1].wait()


@functools.partial(jax.jit)
def kernel(inputs, embed_0, embed_1, embed_2):
    table = jnp.concatenate([embed_0, embed_1, embed_2], axis=0)

    mesh = plsc.VectorSubcoreMesh(core_axis_name="c", subcore_axis_name="s")
    out = pl.kernel(
        _sc_body,
        mesh=mesh,
        compiler_params=pltpu.CompilerParams(
            use_tc_tiling_on_sc=True, needs_layout_passes=False
        ),
        out_type=jax.ShapeDtypeStruct((_F, _B), jnp.float32),
        scratch_types=[
            pltpu.VMEM((_BRW, 3), jnp.int32),
            pltpu.VMEM((_BRW, 3), jnp.int32),
            pltpu.VMEM((_F, _BRW), jnp.float32),
            pltpu.VMEM((_F, _BRW), jnp.float32),
            pltpu.VMEM((_TR, _D), jnp.float32),
            pltpu.SemaphoreType.DMA,
            pltpu.SemaphoreType.DMA,
            pltpu.SemaphoreType.DMA,
            pltpu.SemaphoreType.DMA,
        ],
    )(inputs, table)
    return out.T
